# Initial kernel scaffold; baseline (speedup 1.0000x reference)
import jax
import jax.numpy as jnp
from jax.experimental import pallas as pl


def kernel(query, key_memory, value_memory, k):
    raise NotImplementedError("write your pallas kernel here")


# speedup vs baseline: 1.6233x; 1.6233x over previous
import functools

import jax
import jax.numpy as jnp
from jax import lax
from jax.experimental import pallas as pl
from jax.experimental.pallas import tpu as pltpu
from jax.experimental.pallas import tpu_sc as plsc

Q = 1024
K = 100000
D = 64
TOPK = 10

QT = 128
BK = 2048
NB = (K + BK - 1) // BK
CW = 16
NC = NB * CW

NEG_INF = float("-inf")
I32_MAX = 2**31 - 1


def _score_topk_kernel(q_ref, k_ref, sc_ref, ix_ref):
    bi = pl.program_id(1)
    q = q_ref[...]
    kb = k_ref[...]
    s = lax.dot_general(q, kb, (((1,), (1,)), ((), ())),
                        preferred_element_type=jnp.float32)
    col = lax.broadcasted_iota(jnp.int32, (QT, BK), 1)
    gcol = col + bi * BK
    s = jnp.where(gcol < K, s, NEG_INF)

    best_s, best_i = [], []
    for _ in range(TOPK):
        m = jnp.max(s, axis=1, keepdims=True)
        hit = s == m
        a = jnp.min(jnp.where(hit, col, I32_MAX), axis=1, keepdims=True)
        best_s.append(m)
        best_i.append(a + bi * BK)
        s = jnp.where(col == a, NEG_INF, s)

    pad_s = jnp.full((QT, CW - TOPK), NEG_INF, jnp.float32)
    pad_i = jnp.full((QT, CW - TOPK), I32_MAX, jnp.int32)
    sc_ref[0] = jnp.concatenate(best_s + [pad_s], axis=1)
    ix_ref[0] = jnp.concatenate(best_i + [pad_i], axis=1)


def _merge_kernel(sc_ref, ix_ref, os_ref, oi_ref):
    c = jnp.concatenate([sc_ref[b] for b in range(NB)], axis=1)
    g = jnp.concatenate([ix_ref[b] for b in range(NB)], axis=1)
    out_s, out_i = [], []
    for _ in range(TOPK):
        m = jnp.max(c, axis=1, keepdims=True)
        hit = c == m
        a = jnp.min(jnp.where(hit, g, I32_MAX), axis=1, keepdims=True)
        out_s.append(m)
        out_i.append(a)
        c = jnp.where(hit & (g == a), NEG_INF, c)
    pad_s = jnp.full((QT, CW - TOPK), NEG_INF, jnp.float32)
    pad_i = jnp.zeros((QT, CW - TOPK), jnp.int32)
    os_ref[...] = jnp.concatenate(out_s + [pad_s], axis=1)
    oi_ref[...] = jnp.concatenate(out_i + [pad_i], axis=1)


def _topk_scores_indices(query, key_memory):
    nq = Q // QT
    sc, ix = pl.pallas_call(
        _score_topk_kernel,
        grid=(nq, NB),
        in_specs=[
            pl.BlockSpec((QT, D), lambda qi, bi: (qi, 0)),
            pl.BlockSpec((BK, D), lambda qi, bi: (bi, 0)),
        ],
        out_specs=[
            pl.BlockSpec((1, QT, CW), lambda qi, bi: (bi, qi, 0)),
            pl.BlockSpec((1, QT, CW), lambda qi, bi: (bi, qi, 0)),
        ],
        out_shape=[
            jax.ShapeDtypeStruct((NB, Q, CW), jnp.float32),
            jax.ShapeDtypeStruct((NB, Q, CW), jnp.int32),
        ],
    )(query, key_memory)

    ts, ti = pl.pallas_call(
        _merge_kernel,
        grid=(nq,),
        in_specs=[
            pl.BlockSpec((NB, QT, CW), lambda qi: (0, qi, 0)),
            pl.BlockSpec((NB, QT, CW), lambda qi: (0, qi, 0)),
        ],
        out_specs=[
            pl.BlockSpec((QT, CW), lambda qi: (qi, 0)),
            pl.BlockSpec((QT, CW), lambda qi: (qi, 0)),
        ],
        out_shape=[
            jax.ShapeDtypeStruct((Q, CW), jnp.float32),
            jax.ShapeDtypeStruct((Q, CW), jnp.int32),
        ],
    )(sc, ix)
    return ts[:, :TOPK], ti[:, :TOPK]


def _sc_gather(table, idx):
    info = plsc.get_sparse_core_info()
    nw = info.num_cores * info.num_subcores
    b = idx.shape[0]
    bpw = b // nw
    mesh = plsc.VectorSubcoreMesh(core_axis_name="c", subcore_axis_name="s")

    @functools.partial(
        pl.kernel,
        mesh=mesh,
        out_type=jax.ShapeDtypeStruct((b, D), jnp.float32),
        compiler_params=pltpu.CompilerParams(use_tc_tiling_on_sc=False),
        scratch_types=[
            pltpu.VMEM((bpw,), jnp.int32),
            pltpu.VMEM((bpw, D), jnp.float32),
            pltpu.SemaphoreType.DMA,
        ],
    )
    def gather_k(table_hbm, idx_hbm, out_hbm, idx_v, rows_v, sem):
        wid = lax.axis_index("s") * info.num_cores + lax.axis_index("c")
        base = wid * bpw
        pltpu.sync_copy(idx_hbm.at[pl.ds(base, bpw)], idx_v)
        pltpu.async_copy(table_hbm.at[idx_v], rows_v, sem).wait()
        pltpu.sync_copy(rows_v, out_hbm.at[pl.ds(base, bpw)])

    return gather_k(table, idx)


def kernel(query, key_memory, value_memory, k):
    scores, indices = _topk_scores_indices(query, key_memory)
    rows = _sc_gather(value_memory, indices.reshape(-1))
    return rows.reshape(Q, TOPK, D), scores


# speedup vs baseline: 3.0772x; 1.8957x over previous
import functools

import jax
import jax.numpy as jnp
from jax import lax
from jax.experimental import pallas as pl
from jax.experimental.pallas import tpu as pltpu
from jax.experimental.pallas import tpu_sc as plsc

Q = 1024
K = 100000
D = 64
TOPK = 10

QT = 128
BK = 2048
NB = (K + BK - 1) // BK
KPAD = NB * BK
SB = 256
SPB = BK // SB
NSB = KPAD // SB
CW = 16
GC = TOPK * SB

NEG_INF = float("-inf")
I32_MAX = 2**31 - 1


def _score_max_kernel(q_ref, k_ref, s_ref, mx_ref):
    bi = pl.program_id(0)
    q = q_ref[...]
    kb = k_ref[...]
    s = lax.dot_general(q, kb, (((1,), (1,)), ((), ())),
                        preferred_element_type=jnp.float32)
    col = lax.broadcasted_iota(jnp.int32, (QT, BK), 1)
    s = jnp.where(col + bi * BK < K, s, NEG_INF)
    s_ref[...] = s
    mx_ref[0] = jnp.concatenate(
        [jnp.max(s[:, j * SB:(j + 1) * SB], axis=1, keepdims=True)
         for j in range(SPB)], axis=1)


def _select_blocks_kernel(mx_ref, sb_ref, row_ref):
    qi = pl.program_id(0)
    c = jnp.concatenate([mx_ref[b] for b in range(NB)], axis=1)
    g = lax.broadcasted_iota(jnp.int32, (QT, NSB), 1)
    qrow = (lax.broadcasted_iota(jnp.int32, (QT, 1), 0) + qi * QT) * NSB
    out_sb, out_row = [], []
    for _ in range(TOPK):
        m = jnp.max(c, axis=1, keepdims=True)
        hit = c == m
        a = jnp.min(jnp.where(hit, g, I32_MAX), axis=1, keepdims=True)
        out_sb.append(a)
        out_row.append(qrow + a)
        c = jnp.where(g == a, NEG_INF, c)
    pad = jnp.zeros((QT, CW - TOPK), jnp.int32)
    sb_ref[...] = jnp.concatenate(out_sb + [pad], axis=1)
    row_ref[...] = jnp.concatenate(out_row + [pad], axis=1)


def _final_topk_kernel(gs_ref, sb_ref, os_ref, oi_ref):
    c = gs_ref[...]
    sb = sb_ref[...]
    off = lax.broadcasted_iota(jnp.int32, (QT, SB), 1)
    g = jnp.concatenate(
        [sb[:, j:j + 1] * SB + off for j in range(TOPK)], axis=1)
    out_s, out_i = [], []
    for _ in range(TOPK):
        m = jnp.max(c, axis=1, keepdims=True)
        hit = c == m
        a = jnp.min(jnp.where(hit, g, I32_MAX), axis=1, keepdims=True)
        out_s.append(m)
        out_i.append(a)
        c = jnp.where(hit & (g == a), NEG_INF, c)
    pad_s = jnp.full((QT, CW - TOPK), NEG_INF, jnp.float32)
    pad_i = jnp.zeros((QT, CW - TOPK), jnp.int32)
    os_ref[...] = jnp.concatenate(out_s + [pad_s], axis=1)
    oi_ref[...] = jnp.concatenate(out_i + [pad_i], axis=1)


def _sc_gather(table, idx, width):
    info = plsc.get_sparse_core_info()
    nw = info.num_cores * info.num_subcores
    b = idx.shape[0]
    bpw = b // nw
    mesh = plsc.VectorSubcoreMesh(core_axis_name="c", subcore_axis_name="s")

    @functools.partial(
        pl.kernel,
        mesh=mesh,
        out_type=jax.ShapeDtypeStruct((b, width), jnp.float32),
        compiler_params=pltpu.CompilerParams(use_tc_tiling_on_sc=False),
        scratch_types=[
            pltpu.VMEM((bpw,), jnp.int32),
            pltpu.VMEM((bpw, width), jnp.float32),
            pltpu.SemaphoreType.DMA,
        ],
    )
    def gather_k(table_hbm, idx_hbm, out_hbm, idx_v, rows_v, sem):
        wid = lax.axis_index("s") * info.num_cores + lax.axis_index("c")
        base = wid * bpw
        pltpu.sync_copy(idx_hbm.at[pl.ds(base, bpw)], idx_v)
        pltpu.async_copy(table_hbm.at[idx_v], rows_v, sem).wait()
        pltpu.sync_copy(rows_v, out_hbm.at[pl.ds(base, bpw)])

    return gather_k(table, idx)


def kernel(query, key_memory, value_memory, k):
    nq = Q // QT
    scores, mx = pl.pallas_call(
        _score_max_kernel,
        grid=(NB, nq),
        in_specs=[
            pl.BlockSpec((QT, D), lambda bi, qi: (qi, 0)),
            pl.BlockSpec((BK, D), lambda bi, qi: (bi, 0)),
        ],
        out_specs=[
            pl.BlockSpec((QT, BK), lambda bi, qi: (qi, bi)),
            pl.BlockSpec((1, QT, SPB), lambda bi, qi: (bi, qi, 0)),
        ],
        out_shape=[
            jax.ShapeDtypeStruct((Q, KPAD), jnp.float32),
            jax.ShapeDtypeStruct((NB, Q, SPB), jnp.float32),
        ],
    )(query, key_memory)

    sb, rows = pl.pallas_call(
        _select_blocks_kernel,
        grid=(nq,),
        in_specs=[pl.BlockSpec((NB, QT, SPB), lambda qi: (0, qi, 0))],
        out_specs=[
            pl.BlockSpec((QT, CW), lambda qi: (qi, 0)),
            pl.BlockSpec((QT, CW), lambda qi: (qi, 0)),
        ],
        out_shape=[
            jax.ShapeDtypeStruct((Q, CW), jnp.int32),
            jax.ShapeDtypeStruct((Q, CW), jnp.int32),
        ],
    )(mx)

    gathered = _sc_gather(scores.reshape(Q * NSB, SB),
                          rows[:, :TOPK].reshape(-1), SB)

    ts, ti = pl.pallas_call(
        _final_topk_kernel,
        grid=(nq,),
        in_specs=[
            pl.BlockSpec((QT, GC), lambda qi: (qi, 0)),
            pl.BlockSpec((QT, CW), lambda qi: (qi, 0)),
        ],
        out_specs=[
            pl.BlockSpec((QT, CW), lambda qi: (qi, 0)),
            pl.BlockSpec((QT, CW), lambda qi: (qi, 0)),
        ],
        out_shape=[
            jax.ShapeDtypeStruct((Q, CW), jnp.float32),
            jax.ShapeDtypeStruct((Q, CW), jnp.int32),
        ],
    )(gathered.reshape(Q, GC), sb)

    rows_out = _sc_gather(value_memory, ti[:, :TOPK].reshape(-1), D)
    return rows_out.reshape(Q, TOPK, D), ts[:, :TOPK]
